# Initial kernel scaffold; baseline (speedup 1.0000x reference)
import jax
import jax.numpy as jnp
from jax.experimental import pallas as pl


def kernel(x, table):
    raise NotImplementedError("write your pallas kernel here")


# speedup vs baseline: 11.0378x; 11.0378x over previous
import functools

import jax
import jax.numpy as jnp
from jax import lax
from jax.experimental import pallas as pl
from jax.experimental.pallas import tpu as pltpu
from jax.experimental.pallas import tpu_sc as plsc

CHUNK = 128
NB = 4


@functools.partial(jax.jit, static_argnums=(2, 3))
def _emb_lookup(idx2d, table, nrow, d):
    info = plsc.get_sparse_core_info()
    nc, ns = info.num_cores, info.num_subcores
    nw = nc * ns
    rows_per_worker = nrow // nw
    groups = rows_per_worker // NB
    v = table.shape[0]

    mesh = plsc.VectorSubcoreMesh(core_axis_name="c", subcore_axis_name="s")

    @functools.partial(
        pl.kernel,
        mesh=mesh,
        out_type=jax.ShapeDtypeStruct((nrow, CHUNK, d), jnp.float32),
        scratch_types=[
            pltpu.VMEM_SHARED((v, d), jnp.float32),
            pltpu.VMEM((NB, CHUNK), jnp.int32),
            pltpu.VMEM((NB, CHUNK, d), jnp.float32),
            pltpu.SemaphoreType.DMA,
        ],
    )
    def body(idx_hbm, table_hbm, out_hbm, table_sp, idx_v, rows_v, sem):
        cid = lax.axis_index("c")
        sid = lax.axis_index("s")
        wid = sid * nc + cid

        @pl.when(sid == 0)
        def _():
            pltpu.sync_copy(table_hbm, table_sp)

        plsc.subcore_barrier()

        base_row = wid * rows_per_worker

        def group(g, carry):
            row0 = base_row + g * NB
            pltpu.sync_copy(idx_hbm.at[pl.ds(row0, NB)], idx_v)
            copies = [
                pltpu.async_copy(table_sp.at[idx_v.at[j]], rows_v.at[j], sem)
                for j in range(NB)
            ]
            for cp in copies:
                cp.wait()
            pltpu.sync_copy(rows_v, out_hbm.at[pl.ds(row0, NB)])
            return carry

        lax.fori_loop(0, groups, group, 0)

    return body(idx2d, table)


def kernel(x, table):
    r, c = x.shape
    v, d = table.shape
    b = r * c
    nrow = b // CHUNK
    idx2d = x.reshape(nrow, CHUNK).astype(jnp.int32)
    out = _emb_lookup(idx2d, table, nrow, d)
    return out.reshape(r, c, d)


# speedup vs baseline: 19.8444x; 1.7979x over previous
import functools

import jax
import jax.numpy as jnp
from jax import lax
from jax.experimental import pallas as pl
from jax.experimental.pallas import tpu as pltpu
from jax.experimental.pallas import tpu_sc as plsc

CHUNK = 128
RING = 4


@functools.partial(jax.jit, static_argnums=(2, 3))
def _emb_lookup(idx2d, table, nrow, d):
    info = plsc.get_sparse_core_info()
    nc, ns = info.num_cores, info.num_subcores
    nw = nc * ns
    rows_per_worker = nrow // nw
    nblocks = rows_per_worker // RING
    assert nrow % nw == 0 and rows_per_worker % RING == 0 and nblocks % 2 == 0
    v = table.shape[0]

    mesh = plsc.VectorSubcoreMesh(core_axis_name="c", subcore_axis_name="s")

    @functools.partial(
        pl.kernel,
        mesh=mesh,
        out_type=jax.ShapeDtypeStruct((nrow, CHUNK, d), jnp.float32),
        scratch_types=[
            pltpu.VMEM_SHARED((v, d), jnp.float32),
            pltpu.VMEM((2, RING, CHUNK), jnp.int32),
            pltpu.VMEM((RING, CHUNK, d), jnp.float32),
            pltpu.SemaphoreType.DMA,
            pltpu.SemaphoreType.DMA,
            pltpu.SemaphoreType.DMA,
        ],
    )
    def body(idx_hbm, table_hbm, out_hbm, table_sp, idx_v, rows_v, isem, gsem, wsem):
        cid = lax.axis_index("c")
        sid = lax.axis_index("s")
        wid = sid * nc + cid

        @pl.when(sid == 0)
        def _():
            pltpu.sync_copy(table_hbm, table_sp)

        plsc.subcore_barrier()

        base_row = wid * rows_per_worker
        last_blk_row = base_row + rows_per_worker - RING

        def fire_idx(p, q):
            row = lax.min(base_row + p * RING, last_blk_row)
            pltpu.async_copy(idx_hbm.at[pl.ds(row, RING)], idx_v.at[q], isem)

        def drain_idx(q):
            pltpu.make_async_copy(
                idx_hbm.at[pl.ds(base_row, RING)], idx_v.at[q], isem
            ).wait()

        def fire_gather(q, b):
            pltpu.async_copy(table_sp.at[idx_v.at[q].at[b]], rows_v.at[b], gsem)

        def drain_gather(b):
            pltpu.make_async_copy(
                table_hbm.at[pl.ds(0, CHUNK)], rows_v.at[b], gsem
            ).wait()

        def fire_write(row, b):
            pltpu.async_copy(rows_v.at[b], out_hbm.at[row], wsem)

        def drain_write(b):
            pltpu.make_async_copy(rows_v.at[b], out_hbm.at[0], wsem).wait()

        def block(p, q, first):
            if not first:
                drain_idx(q)
            for b in range(RING):
                if not first:
                    drain_write(b)
                fire_gather(q, b)
                if not (first and b < 2):
                    bb = (b + 2) % RING
                    drain_gather(bb)
                    fire_write(base_row + p * RING + bb - (RING if b < 2 else 0), bb)
                if b == 1:
                    fire_idx(p + 1, 1 - q)

        pltpu.sync_copy(idx_hbm.at[pl.ds(base_row, RING)], idx_v.at[0])
        block(0, 0, True)

        def pair(pp, carry):
            block(2 * pp + 1, 1, False)
            block(2 * pp + 2, 0, False)
            return carry

        lax.fori_loop(0, (nblocks - 2) // 2, pair, 0)
        block(nblocks - 1, 1, False)

        drain_idx(0)
        last_row = base_row + rows_per_worker - RING
        for b in (2, 3):
            drain_gather(b)
            fire_write(last_row + b, b)
        for b in range(RING):
            drain_write(b)

    return body(idx2d, table)


def kernel(x, table):
    r, c = x.shape
    v, d = table.shape
    b = r * c
    nrow = b // CHUNK
    idx2d = x.reshape(nrow, CHUNK).astype(jnp.int32)
    out = _emb_lookup(idx2d, table, nrow, d)
    return out.reshape(r, c, d)
